# baseline (speedup 1.0000x reference)
import functools

import jax
import jax.numpy as jnp
from jax import lax
from jax.experimental import pallas as pl
from jax.experimental.pallas import tpu as pltpu
from jax.experimental.pallas import tpu_sc as plsc

N = 10000
D = 128
G = 64

NC = 2
NS = 16
NW = NC * NS
K = 128
K2 = 64
NPAD = 10240
RPT = NPAD // NS
RC = RPT // K
RC2 = RPT // K2


@functools.cache
def _mesh():
    return plsc.VectorSubcoreMesh(core_axis_name="c", subcore_axis_name="s",
                                  num_cores=NC, num_subcores=NS)


def _zero_vmem(buf, rows, width):
    def body(i, _):
        for j in range(width // 16):
            buf[i, pl.ds(j * 16, 16)] = jnp.zeros((16,), jnp.float32)
        return 0
    lax.fori_loop(0, rows, body, 0)


def _sc_deg_body(dst_hbm, out_hbm, dsts, ones_v, acc_sh, semi, semd):
    c = lax.axis_index("c")
    s = lax.axis_index("s")
    wid = c * NS + s
    nch = dst_hbm.shape[1]
    cpi = pltpu.async_copy(dst_hbm.at[wid], dsts, semi)
    _zero_vmem(ones_v, K, D)
    for t in range(RC):
        pltpu.sync_copy(ones_v, acc_sh.at[pl.ds(s * RPT + t * K, K)])
    cpi.wait()
    plsc.subcore_barrier()
    def fill(i, _):
        for j in range(D // 16):
            ones_v[i, pl.ds(j * 16, 16)] = jnp.ones((16,), jnp.float32)
        return 0
    lax.fori_loop(0, K, fill, 0)
    for i in range(4):
        pltpu.async_copy(ones_v, acc_sh.at[dsts.at[i]], semd, add=True)
    def body(i, _):
        pltpu.make_async_copy(ones_v, acc_sh.at[dsts.at[0]], semd).wait()
        pltpu.async_copy(ones_v, acc_sh.at[dsts.at[i + 4]], semd, add=True)
        return 0
    lax.fori_loop(0, nch - 4, body, 0)
    for i in range(4):
        pltpu.make_async_copy(ones_v, acc_sh.at[dsts.at[0]], semd).wait()
    plsc.subcore_barrier()
    for t in range(RC):
        r = s * RPT + t * K
        pltpu.sync_copy(acc_sh.at[pl.ds(r, K)], ones_v)
        pltpu.sync_copy(ones_v, out_hbm.at[c, pl.ds(r, K)])


@functools.cache
def _sc_deg_kernel(nch):
    return pl.kernel(
        _sc_deg_body, mesh=_mesh(),
        out_type=jax.ShapeDtypeStruct((NC, NPAD, D), jnp.float32),
        scratch_types=[
            pltpu.VMEM((nch, K), jnp.int32),
            pltpu.VMEM((K, D), jnp.float32),
            pltpu.VMEM_SHARED((NPAD, D), jnp.float32),
            pltpu.SemaphoreType.DMA,
            pltpu.SemaphoreType.DMA,
        ],
    )


def _sc_deg(dst3):
    return _sc_deg_kernel(dst3.shape[1])(dst3)


@functools.cache
def _sc_agg_kernel(nch_a, nch_b):
    def body_fn(ts_hbm, src_hbm, dst_hbm, out_hbm, srcs, dsts, b0, b1, b2,
                b3, fbuf, acc_sh, g0, g1, g2, g3, ssem, semi):
        c = lax.axis_index("c")
        s = lax.axis_index("s")
        bfs = (b0, b1, b2, b3)
        gsem = (g0, g1, g2, g3)

        def gather(i, o):
            pltpu.async_copy(ts_hbm.at[srcs.at[i]], bfs[o], gsem[o])

        def wait_g(o):
            pltpu.make_async_copy(ts_hbm.at[srcs.at[0]], bfs[o],
                                  gsem[o]).wait()

        def scatter(i):
            pltpu.async_copy(fbuf, acc_sh.at[dsts.at[i]], ssem, add=True)

        def wait_s():
            pltpu.make_async_copy(fbuf, acc_sh.at[dsts.at[0]], ssem).wait()

        def conv(o):
            def cbody(r, _):
                for j in range(D // 32):
                    v = bfs[o][r, pl.ds(j * 16, 16)]
                    lo = lax.bitcast_convert_type(v << 16, jnp.float32)
                    hi = lax.bitcast_convert_type(v & jnp.int32(-65536), jnp.float32)
                    fbuf[r, pl.ds(j * 32, 16)] = lo
                    fbuf[r, pl.ds(j * 32 + 16, 16)] = hi
                return 0
            lax.fori_loop(0, K2, cbody, 0)

        def run(base_ch, nch):
            qch = 32
            for h in range(nch // qch):
                b = base_ch + h * qch
                pltpu.async_copy(src_hbm.at[pl.ds(b, qch)],
                                 srcs.at[pl.ds(0, qch)], semi)
                pltpu.async_copy(dst_hbm.at[pl.ds(b, qch)],
                                 dsts.at[pl.ds(0, qch)], semi)
                pltpu.make_async_copy(src_hbm.at[pl.ds(0, qch)],
                                      srcs.at[pl.ds(0, qch)], semi).wait()
                pltpu.make_async_copy(dst_hbm.at[pl.ds(0, qch)],
                                      dsts.at[pl.ds(0, qch)], semi).wait()

                def sub(i, o, first):
                    gather(i, o)
                    wait_g((o + 1) % 4)
                    if not first:
                        wait_s()
                    conv((o + 1) % 4)
                    scatter(i - 3)

                gather(0, 0)
                gather(1, 1)
                gather(2, 2)
                sub(3, 3, True)

                def lbody(p, _):
                    for o4 in range(4):
                        sub(4 + 4 * p + o4, o4, False)
                    return 0

                lax.fori_loop(0, (qch - 4) // 4, lbody, 0)
                for i in (qch - 3, qch - 2, qch - 1):
                    wait_g(i % 4)
                    wait_s()
                    conv(i % 4)
                    scatter(i)
                wait_s()

        _zero_vmem(fbuf, K2, D)
        for t in range(RC2):
            pltpu.sync_copy(fbuf, acc_sh.at[pl.ds(s * RPT + t * K2, K2)])
        plsc.subcore_barrier()

        @pl.when(c == 0)
        def _():
            run(s * nch_a, nch_a)

        @pl.when(c == 1)
        def _():
            run(NS * nch_a + s * nch_b, nch_b)

        plsc.subcore_barrier()
        for t in range(RC2):
            r = s * RPT + t * K2
            pltpu.sync_copy(acc_sh.at[pl.ds(r, K2)], fbuf)
            pltpu.sync_copy(fbuf, out_hbm.at[c, pl.ds(r, K2)])

    mq = 32
    return pl.kernel(
        body_fn, mesh=_mesh(),
        compiler_params=pltpu.CompilerParams(use_tc_tiling_on_sc=False),
        out_type=jax.ShapeDtypeStruct((NC, NPAD, D), jnp.float32),
        scratch_types=[
            pltpu.VMEM((mq, K2), jnp.int32),
            pltpu.VMEM((mq, K2), jnp.int32),
            pltpu.VMEM((K2, D // 2), jnp.int32),
            pltpu.VMEM((K2, D // 2), jnp.int32),
            pltpu.VMEM((K2, D // 2), jnp.int32),
            pltpu.VMEM((K2, D // 2), jnp.int32),
            pltpu.VMEM((K2, D), jnp.float32),
            pltpu.VMEM_SHARED((NPAD, D), jnp.float32),
            pltpu.SemaphoreType.DMA,
            pltpu.SemaphoreType.DMA,
            pltpu.SemaphoreType.DMA,
            pltpu.SemaphoreType.DMA,
            pltpu.SemaphoreType.DMA,
            pltpu.SemaphoreType.DMA,
        ],
    )


def _sc_agg(ts_bf, src2, dst2, nch_a, nch_b):
    return _sc_agg_kernel(nch_a, nch_b)(ts_bf, src2, dst2)


def _dinv_block(deg_ref):
    d = deg_ref[0][:, 0:1] + deg_ref[1][:, 0:1]
    return lax.rsqrt(d + 1.0)


RB = 1000


def _tc_first_body(x_ref, w_ref, deg_ref, out_ref):
    dinv = _dinv_block(deg_ref)
    out_ref[...] = jnp.dot(x_ref[...], w_ref[...],
                           preferred_element_type=jnp.float32) * dinv


def _tc_mid_body(p_ref, ts_ref, deg_ref, b_ref, w_ref, out_ref):
    dinv = _dinv_block(deg_ref)
    h = jnp.maximum((p_ref[0] + p_ref[1] + ts_ref[...]) * dinv + b_ref[...],
                    0.0)
    out_ref[...] = jnp.dot(h, w_ref[...],
                           preferred_element_type=jnp.float32) * dinv


def _tc_final_body(p_ref, ts_ref, deg_ref, b_ref, batch_ref, lw_ref, lb_ref,
                   hg_ref, lp_ref, sums, cnts):
    i = pl.program_id(0)

    @pl.when(i == 0)
    def _():
        sums[...] = jnp.zeros_like(sums)
        cnts[...] = jnp.zeros_like(cnts)

    dinv = _dinv_block(deg_ref)
    h = jnp.maximum((p_ref[0] + p_ref[1] + ts_ref[...]) * dinv + b_ref[...],
                    0.0)
    ids = lax.broadcasted_iota(jnp.int32, (G, RB), 0)
    mask = (ids == jnp.broadcast_to(batch_ref[0], (G, RB))).astype(
        jnp.float32)
    sums[...] += jnp.dot(mask, h, preferred_element_type=jnp.float32)
    cnts[...] += jnp.broadcast_to(jnp.sum(mask, axis=1, keepdims=True),
                                  (G, D))

    @pl.when(i == pl.num_programs(0) - 1)
    def _():
        hg = sums[...] / jnp.maximum(cnts[...], 1.0)
        hg_ref[...] = hg
        logits = jnp.dot(hg, lw_ref[...],
                         preferred_element_type=jnp.float32) + lb_ref[...]
        m = jnp.max(logits, axis=1, keepdims=True)
        lse = jnp.log(jnp.sum(jnp.exp(logits - m), axis=1, keepdims=True)) + m
        lp_ref[...] = logits - lse


_GRID = N // RB

_deg_spec = pl.BlockSpec((NC, RB, D), lambda i: (0, i, 0))
_p_spec = pl.BlockSpec((NC, RB, D), lambda i: (0, i, 0))
_row_spec = pl.BlockSpec((RB, D), lambda i: (i, 0))
_w_spec = pl.BlockSpec((D, D), lambda i: (0, 0))
_b_spec = pl.BlockSpec((1, D), lambda i: (0, 0))


def _tc_first(x, w0, deg):
    return pl.pallas_call(
        _tc_first_body,
        grid=(_GRID,),
        in_specs=[_row_spec, _w_spec, _deg_spec],
        out_specs=_row_spec,
        out_shape=jax.ShapeDtypeStruct((N, D), jnp.float32),
    )(x, w0, deg)


def _tc_mid(parts, ts, deg, b, w):
    return pl.pallas_call(
        _tc_mid_body,
        grid=(_GRID,),
        in_specs=[_p_spec, _row_spec, _deg_spec, _b_spec, _w_spec],
        out_specs=_row_spec,
        out_shape=jax.ShapeDtypeStruct((N, D), jnp.float32),
    )(parts, ts, deg, b, w)


def _tc_final(parts, ts, deg, b, batch3d, lw, lb):
    return pl.pallas_call(
        _tc_final_body,
        grid=(_GRID,),
        in_specs=[
            _p_spec, _row_spec, _deg_spec, _b_spec,
            pl.BlockSpec((1, 1, RB), lambda i: (i, 0, 0)),
            pl.BlockSpec((D, 16), lambda i: (0, 0)),
            pl.BlockSpec((1, 16), lambda i: (0, 0)),
        ],
        out_specs=[
            pl.BlockSpec((G, D), lambda i: (0, 0)),
            pl.BlockSpec((G, 16), lambda i: (0, 0)),
        ],
        out_shape=[
            jax.ShapeDtypeStruct((G, D), jnp.float32),
            jax.ShapeDtypeStruct((G, 16), jnp.float32),
        ],
        scratch_shapes=[
            pltpu.VMEM((G, D), jnp.float32),
            pltpu.VMEM((G, D), jnp.float32),
        ],
    )(parts, ts, deg, b, batch3d, lw, lb)


import numpy as _np

_BF_PERM = tuple(
    _np.arange(D).reshape(D // 32, 2, 16).transpose(0, 2, 1).reshape(D)
)


def _to_bf(ts):
    tb = ts.astype(jnp.bfloat16)[:, jnp.array(_BF_PERM)]
    return jax.lax.bitcast_convert_type(tb.reshape(ts.shape[0], D // 2, 2),
                                        jnp.int32)


def kernel(x, edge_index, batch, num_layers, W0, b0, W1, b1, W2, b2,
           lin_W, lin_b):
    src = edge_index[0].astype(jnp.int32)
    dst = edge_index[1].astype(jnp.int32)
    e = src.shape[0]
    nch_a, nch_b = 256, 64
    while NS * (nch_a + nch_b) * K2 < e:
        nch_a += 96
    tot = NS * (nch_a + nch_b)
    epad = tot * K2
    src_p = jnp.concatenate([src, jnp.zeros((epad - e,), jnp.int32)])
    dst_p = jnp.concatenate([dst, jnp.full((epad - e,), N, jnp.int32)])
    src2 = src_p.reshape(tot, K2)
    dst2 = dst_p.reshape(tot, K2)

    deg = _sc_deg(dst_p.reshape(NW, epad // (NW * K), K))
    ts = _tc_first(x, W0, deg)
    for b_prev, w_next in ((b0.reshape(1, D), W1), (b1.reshape(1, D), W2)):
        parts = _sc_agg(_to_bf(ts), src2, dst2, nch_a, nch_b)
        ts = _tc_mid(parts, ts, deg, b_prev, w_next)
    parts = _sc_agg(_to_bf(ts), src2, dst2, nch_a, nch_b)
    hg, lp = _tc_final(parts, ts, deg, b2.reshape(1, D),
                       batch.astype(jnp.int32).reshape(_GRID, 1, RB),
                       lin_W, lin_b.reshape(1, 16))
    return (hg, lp)


# speedup vs baseline: 1.1888x; 1.1888x over previous
import functools

import jax
import jax.numpy as jnp
from jax import lax
from jax.experimental import pallas as pl
from jax.experimental.pallas import tpu as pltpu
from jax.experimental.pallas import tpu_sc as plsc

N = 10000
D = 128
G = 64

NC = 2
NS = 16
NW = NC * NS
K = 128
K2 = 64
NPAD = 10240
RPT = NPAD // NS
RC = RPT // K
RC2 = RPT // K2


@functools.cache
def _mesh():
    return plsc.VectorSubcoreMesh(core_axis_name="c", subcore_axis_name="s",
                                  num_cores=NC, num_subcores=NS)


def _zero_vmem(buf, rows, width):
    def body(i, _):
        for j in range(width // 16):
            buf[i, pl.ds(j * 16, 16)] = jnp.zeros((16,), jnp.float32)
        return 0
    lax.fori_loop(0, rows, body, 0)


def _sc_deg_body(dst_hbm, out_hbm, dsts, ones_v, acc_sh, semi, semd):
    c = lax.axis_index("c")
    s = lax.axis_index("s")
    wid = c * NS + s
    nch = dst_hbm.shape[1]
    cpi = pltpu.async_copy(dst_hbm.at[wid], dsts, semi)
    _zero_vmem(ones_v, K, D)
    for t in range(RC):
        pltpu.sync_copy(ones_v, acc_sh.at[pl.ds(s * RPT + t * K, K)])
    cpi.wait()
    plsc.subcore_barrier()
    def fill(i, _):
        for j in range(D // 16):
            ones_v[i, pl.ds(j * 16, 16)] = jnp.ones((16,), jnp.float32)
        return 0
    lax.fori_loop(0, K, fill, 0)
    for i in range(4):
        pltpu.async_copy(ones_v, acc_sh.at[dsts.at[i]], semd, add=True)
    def body(i, _):
        pltpu.make_async_copy(ones_v, acc_sh.at[dsts.at[0]], semd).wait()
        pltpu.async_copy(ones_v, acc_sh.at[dsts.at[i + 4]], semd, add=True)
        return 0
    lax.fori_loop(0, nch - 4, body, 0)
    for i in range(4):
        pltpu.make_async_copy(ones_v, acc_sh.at[dsts.at[0]], semd).wait()
    plsc.subcore_barrier()
    for t in range(RC):
        r = s * RPT + t * K
        pltpu.sync_copy(acc_sh.at[pl.ds(r, K)], ones_v)
        pltpu.sync_copy(ones_v, out_hbm.at[c, pl.ds(r, K)])


@functools.cache
def _sc_deg_kernel(nch):
    return pl.kernel(
        _sc_deg_body, mesh=_mesh(),
        out_type=jax.ShapeDtypeStruct((NC, NPAD, D), jnp.float32),
        scratch_types=[
            pltpu.VMEM((nch, K), jnp.int32),
            pltpu.VMEM((K, D), jnp.float32),
            pltpu.VMEM_SHARED((NPAD, D), jnp.float32),
            pltpu.SemaphoreType.DMA,
            pltpu.SemaphoreType.DMA,
        ],
    )


def _sc_deg(dst3):
    return _sc_deg_kernel(dst3.shape[1])(dst3)


@functools.cache
def _sc_agg_kernel(nch):
    def body_fn(ts_hbm, src_hbm, dst_hbm, out_hbm, src_v, dst_v, rows_v,
                acc_sh, sem):
        c = lax.axis_index("c")
        s = lax.axis_index("s")
        wid = c * NS + s
        _zero_vmem(rows_v, K, D)
        for t in range(RC):
            pltpu.sync_copy(rows_v, acc_sh.at[pl.ds(s * RPT + t * K, K)])
        plsc.subcore_barrier()

        def body(i, _):
            pltpu.sync_copy(src_hbm.at[wid, i], src_v)
            pltpu.sync_copy(dst_hbm.at[wid, i], dst_v)
            pltpu.async_copy(ts_hbm.at[src_v], rows_v, sem).wait()
            pltpu.sync_copy(rows_v, acc_sh.at[dst_v], add=True)
            return 0

        lax.fori_loop(0, nch, body, 0)
        plsc.subcore_barrier()
        for t in range(RC):
            r = s * RPT + t * K
            pltpu.sync_copy(acc_sh.at[pl.ds(r, K)], rows_v)
            pltpu.sync_copy(rows_v, out_hbm.at[c, pl.ds(r, K)])

    return pl.kernel(
        body_fn, mesh=_mesh(),
        out_type=jax.ShapeDtypeStruct((NC, NPAD, D), jnp.float32),
        scratch_types=[
            pltpu.VMEM((K,), jnp.int32),
            pltpu.VMEM((K,), jnp.int32),
            pltpu.VMEM((K, D), jnp.float32),
            pltpu.VMEM_SHARED((NPAD, D), jnp.float32),
            pltpu.SemaphoreType.DMA,
        ],
    )


def _sc_agg(ts, src3, dst3):
    return _sc_agg_kernel(src3.shape[1])(ts, src3, dst3)


def _dinv_block(deg_ref):
    d = deg_ref[0][:, 0:1] + deg_ref[1][:, 0:1]
    return lax.rsqrt(d + 1.0)


RB = 1000


def _tc_first_body(x_ref, w_ref, deg_ref, out_ref):
    dinv = _dinv_block(deg_ref)
    out_ref[...] = jnp.dot(x_ref[...], w_ref[...],
                           preferred_element_type=jnp.float32) * dinv


def _tc_mid_body(p_ref, ts_ref, deg_ref, b_ref, w_ref, out_ref):
    dinv = _dinv_block(deg_ref)
    h = jnp.maximum((p_ref[0] + p_ref[1] + ts_ref[...]) * dinv + b_ref[...],
                    0.0)
    out_ref[...] = jnp.dot(h, w_ref[...],
                           preferred_element_type=jnp.float32) * dinv


def _tc_final_body(p_ref, ts_ref, deg_ref, b_ref, batch_ref, lw_ref, lb_ref,
                   hg_ref, lp_ref, sums, cnts):
    i = pl.program_id(0)

    @pl.when(i == 0)
    def _():
        sums[...] = jnp.zeros_like(sums)
        cnts[...] = jnp.zeros_like(cnts)

    dinv = _dinv_block(deg_ref)
    h = jnp.maximum((p_ref[0] + p_ref[1] + ts_ref[...]) * dinv + b_ref[...],
                    0.0)
    ids = lax.broadcasted_iota(jnp.int32, (G, RB), 0)
    mask = (ids == jnp.broadcast_to(batch_ref[0], (G, RB))).astype(
        jnp.float32)
    sums[...] += jnp.dot(mask, h, preferred_element_type=jnp.float32)
    cnts[...] += jnp.broadcast_to(jnp.sum(mask, axis=1, keepdims=True),
                                  (G, D))

    @pl.when(i == pl.num_programs(0) - 1)
    def _():
        hg = sums[...] / jnp.maximum(cnts[...], 1.0)
        hg_ref[...] = hg
        logits = jnp.dot(hg, lw_ref[...],
                         preferred_element_type=jnp.float32) + lb_ref[...]
        m = jnp.max(logits, axis=1, keepdims=True)
        lse = jnp.log(jnp.sum(jnp.exp(logits - m), axis=1, keepdims=True)) + m
        lp_ref[...] = logits - lse


_GRID = N // RB

_deg_spec = pl.BlockSpec((NC, RB, D), lambda i: (0, i, 0))
_p_spec = pl.BlockSpec((NC, RB, D), lambda i: (0, i, 0))
_row_spec = pl.BlockSpec((RB, D), lambda i: (i, 0))
_w_spec = pl.BlockSpec((D, D), lambda i: (0, 0))
_b_spec = pl.BlockSpec((1, D), lambda i: (0, 0))


def _tc_first(x, w0, deg):
    return pl.pallas_call(
        _tc_first_body,
        grid=(_GRID,),
        in_specs=[_row_spec, _w_spec, _deg_spec],
        out_specs=_row_spec,
        out_shape=jax.ShapeDtypeStruct((N, D), jnp.float32),
    )(x, w0, deg)


def _tc_mid(parts, ts, deg, b, w):
    return pl.pallas_call(
        _tc_mid_body,
        grid=(_GRID,),
        in_specs=[_p_spec, _row_spec, _deg_spec, _b_spec, _w_spec],
        out_specs=_row_spec,
        out_shape=jax.ShapeDtypeStruct((N, D), jnp.float32),
    )(parts, ts, deg, b, w)


def _tc_final(parts, ts, deg, b, batch3d, lw, lb):
    return pl.pallas_call(
        _tc_final_body,
        grid=(_GRID,),
        in_specs=[
            _p_spec, _row_spec, _deg_spec, _b_spec,
            pl.BlockSpec((1, 1, RB), lambda i: (i, 0, 0)),
            pl.BlockSpec((D, 16), lambda i: (0, 0)),
            pl.BlockSpec((1, 16), lambda i: (0, 0)),
        ],
        out_specs=[
            pl.BlockSpec((G, D), lambda i: (0, 0)),
            pl.BlockSpec((G, 16), lambda i: (0, 0)),
        ],
        out_shape=[
            jax.ShapeDtypeStruct((G, D), jnp.float32),
            jax.ShapeDtypeStruct((G, 16), jnp.float32),
        ],
        scratch_shapes=[
            pltpu.VMEM((G, D), jnp.float32),
            pltpu.VMEM((G, D), jnp.float32),
        ],
    )(parts, ts, deg, b, batch3d, lw, lb)


def kernel(x, edge_index, batch, num_layers, W0, b0, W1, b1, W2, b2,
           lin_W, lin_b):
    src = edge_index[0].astype(jnp.int32)
    dst = edge_index[1].astype(jnp.int32)
    e = src.shape[0]
    nch = (e + NW * K - 1) // (NW * K)
    epad = NW * K * nch
    src_p = jnp.concatenate([src, jnp.zeros((epad - e,), jnp.int32)])
    dst_p = jnp.concatenate([dst, jnp.full((epad - e,), N, jnp.int32)])
    src3 = src_p.reshape(NW, nch, K)
    dst3 = dst_p.reshape(NW, nch, K)

    deg = _sc_deg(dst3)
    ts = _tc_first(x, W0, deg)
    for b_prev, w_next in ((b0.reshape(1, D), W1), (b1.reshape(1, D), W2)):
        parts = _sc_agg(ts, src3, dst3)
        ts = _tc_mid(parts, ts, deg, b_prev, w_next)
    parts = _sc_agg(ts, src3, dst3)
    hg, lp = _tc_final(parts, ts, deg, b2.reshape(1, D),
                       batch.astype(jnp.int32).reshape(_GRID, 1, RB),
                       lin_W, lin_b.reshape(1, 16))
    return (hg, lp)
